# baseline (speedup 1.0000x reference)
import functools

import jax
import jax.numpy as jnp
from jax import lax
from jax.experimental import pallas as pl
from jax.experimental.pallas import tpu as pltpu
from jax.experimental.pallas import tpu_sc as plsc

B = 16384
L = 4
V = 4
D = 8
TD = 64
LV = L * V
VOCAB = 1000000

CH = 128
TC_C = 2048


def _transpose_body(t_ref, o_ref, y_ref):
    cq = TC_C // 8
    x = t_ref[...].reshape(LV, TC_C)
    jj = lax.broadcasted_iota(jnp.int32, (LV, 128), 0)
    mm = lax.broadcasted_iota(jnp.int32, (LV, 128), 1)
    rep = (jj == mm % LV).astype(jnp.float32)
    y = lax.dot_general(x, rep, (((0,), (0,)), ((), ())),
                        preferred_element_type=jnp.float32,
                        precision=jax.lax.Precision.HIGHEST)
    y_ref[...] = y.reshape(cq, 8, 128)
    grp = lax.broadcasted_iota(jnp.int32, (cq, 128), 1) // LV
    acc = y_ref[:, 0, :]
    for v in range(1, 8):
        acc = jnp.where(grp == v, y_ref[:, v, :], acc)
    o_ref[...] = acc


def _tc_transpose(tt3):
    n_steps = (VOCAB + TC_C - 1) // TC_C
    vpad = n_steps * TC_C
    return pl.pallas_call(
        _transpose_body,
        grid=(n_steps,),
        in_specs=[pl.BlockSpec((L, V, TC_C), lambda i: (0, 0, i))],
        out_specs=pl.BlockSpec((TC_C // 8, 128), lambda i: (i, 0)),
        out_shape=jax.ShapeDtypeStruct((vpad // 8, 128), jnp.float32),
        scratch_shapes=[pltpu.VMEM((TC_C // 8, 8, 128), jnp.float32)],
    )(tt3)


def _sc_gather(table2d, idx3d, n_ch, b_per_w, nc):
    mesh = plsc.VectorSubcoreMesh(core_axis_name="c", subcore_axis_name="s")

    @functools.partial(
        pl.kernel,
        mesh=mesh,
        out_type=jax.ShapeDtypeStruct((B, LV), jnp.float32),
        scratch_types=[
            pltpu.VMEM((n_ch, CH), jnp.int32),
            pltpu.VMEM((b_per_w, LV), jnp.float32),
            pltpu.SemaphoreType.DMA,
        ],
        compiler_params=pltpu.CompilerParams(use_tc_tiling_on_sc=False),
    )
    def k(table_hbm, idx_hbm, out_hbm, idx_v, rows_v, sem):
        wid = lax.axis_index("s") * nc + lax.axis_index("c")
        base = wid * b_per_w
        pltpu.sync_copy(idx_hbm.at[wid], idx_v)
        copies = []
        for j in range(n_ch):
            copies.append(
                pltpu.async_copy(
                    table_hbm.at[idx_v.at[j]],
                    rows_v.at[pl.ds(j * CH, CH)],
                    sem,
                )
            )
        for c in copies:
            c.wait()
        pltpu.sync_copy(rows_v, out_hbm.at[pl.ds(base, b_per_w)])

    return k(table2d, idx3d)


def _dense_body(g_ref, vr_ref, t_ref, w_ref, b_ref, o_ref):
    x = g_ref[...] * t_ref[...]
    m = jnp.max(x, axis=1, keepdims=True)
    e = jnp.exp(x - m)
    ii = lax.broadcasted_iota(jnp.int32, (LV, LV), 0) // V
    jj = lax.broadcasted_iota(jnp.int32, (LV, LV), 1) // V
    gm = (ii == jj).astype(jnp.float32)
    s = jax.lax.dot(e, gm, preferred_element_type=jnp.float32,
                    precision=jax.lax.Precision.HIGHEST)
    w = e / s
    vr = vr_ref[...]
    vt = jnp.concatenate([vr, vr, vr, vr], axis=1)
    ri = lax.broadcasted_iota(jnp.int32, (LV, L * D), 0) // V
    ci = lax.broadcasted_iota(jnp.int32, (LV, L * D), 1) // D
    bd = jnp.where(ri == ci, vt, 0.0)
    mm = lax.dot_general(bd, w_ref[...], (((1,), (1,)), ((), ())),
                         preferred_element_type=jnp.float32,
                         precision=jax.lax.Precision.HIGHEST)
    o_ref[...] = (
        jax.lax.dot(w, mm, preferred_element_type=jnp.float32,
                    precision=jax.lax.Precision.HIGHEST) + b_ref[...]
    )


def _tc_dense(g, vr, t_full, w, b2d):
    blk = 2048
    grid = (B // blk,)
    return pl.pallas_call(
        _dense_body,
        grid=grid,
        in_specs=[
            pl.BlockSpec((blk, LV), lambda i: (i, 0)),
            pl.BlockSpec((LV, D), lambda i: (0, 0)),
            pl.BlockSpec((1, LV), lambda i: (0, 0)),
            pl.BlockSpec((TD, L * D), lambda i: (0, 0)),
            pl.BlockSpec((1, TD), lambda i: (0, 0)),
        ],
        out_specs=pl.BlockSpec((blk, TD), lambda i: (i, 0)),
        out_shape=jax.ShapeDtypeStruct((B, TD), jnp.float32),
    )(g, vr, t_full, w, b2d)


def kernel(idx_batch, logits_table, vertices, logit_temps, W, b):
    info = plsc.get_sparse_core_info()
    nw = info.num_cores * info.num_subcores
    b_per_w = B // nw
    n_ch = b_per_w // CH

    tt3 = logits_table.transpose(1, 2, 0)
    t8 = _tc_transpose(tt3)
    table2d = t8.reshape(-1, LV)

    idx3d = idx_batch.astype(jnp.int32).reshape(nw, n_ch, CH)
    g = _sc_gather(table2d, idx3d, n_ch, b_per_w, info.num_cores)

    vr = vertices.reshape(LV, D)
    t_full = jnp.repeat(logit_temps, V).reshape(1, LV)
    b2d = b.reshape(1, TD)
    return _tc_dense(g, vr, t_full, W, b2d)


# speedup vs baseline: 2.3842x; 2.3842x over previous
import functools

import jax
import jax.numpy as jnp
from jax import lax
from jax.experimental import pallas as pl
from jax.experimental.pallas import tpu as pltpu
from jax.experimental.pallas import tpu_sc as plsc

B = 16384
L = 4
V = 4
D = 8
TD = 64
LV = L * V
VOCAB = 1000000

CH = 128
TC_C = 8192


def _transpose_body(t_ref, o_ref):
    cq = TC_C // 8
    x = t_ref[...].reshape(LV, TC_C)
    lane = lax.broadcasted_iota(jnp.int32, (LV, TC_C), 1)
    gvo = pl.program_id(0) * TC_C + lane
    x = jnp.where(gvo < VOCAB, x, 0.0)
    w = jnp.swapaxes(x, 0, 1)
    for u in range(8):
        o_ref[:, LV * u:LV * (u + 1)] = w[u * cq:(u + 1) * cq, :]


def _tc_transpose(tt3):
    n_steps = (VOCAB + TC_C - 1) // TC_C
    vpad = n_steps * TC_C
    return pl.pallas_call(
        _transpose_body,
        grid=(n_steps,),
        in_specs=[pl.BlockSpec((L, V, TC_C), lambda i: (0, 0, i))],
        out_specs=pl.BlockSpec((TC_C // 8, 128), lambda i: (i, 0)),
        out_shape=jax.ShapeDtypeStruct((vpad // 8, 128), jnp.float32),
    )(tt3)


def _sc_gather(table2d, idx3d, n_ch, b_per_w, nc):
    mesh = plsc.VectorSubcoreMesh(core_axis_name="c", subcore_axis_name="s")

    @functools.partial(
        pl.kernel,
        mesh=mesh,
        out_type=jax.ShapeDtypeStruct((B, LV), jnp.float32),
        scratch_types=[
            pltpu.VMEM((n_ch, CH), jnp.int32),
            pltpu.VMEM((b_per_w, LV), jnp.float32),
            pltpu.SemaphoreType.DMA,
        ],
        compiler_params=pltpu.CompilerParams(use_tc_tiling_on_sc=False),
    )
    def k(table_hbm, idx_hbm, out_hbm, idx_v, rows_v, sem):
        wid = lax.axis_index("s") * nc + lax.axis_index("c")
        base = wid * b_per_w
        pltpu.sync_copy(idx_hbm.at[wid], idx_v)
        copies = []
        for j in range(n_ch):
            copies.append(
                pltpu.async_copy(
                    table_hbm.at[idx_v.at[j]],
                    rows_v.at[pl.ds(j * CH, CH)],
                    sem,
                )
            )
        for c in copies:
            c.wait()
        pltpu.sync_copy(rows_v, out_hbm.at[pl.ds(base, b_per_w)])

    return k(table2d, idx3d)


def _dense_body(g_ref, vr_ref, t_ref, w_ref, b_ref, o_ref):
    x = g_ref[...] * t_ref[...]
    m = jnp.max(x, axis=1, keepdims=True)
    e = jnp.exp(x - m)
    ii = lax.broadcasted_iota(jnp.int32, (LV, LV), 0) // V
    jj = lax.broadcasted_iota(jnp.int32, (LV, LV), 1) // V
    gm = (ii == jj).astype(jnp.float32)
    s = jax.lax.dot(e, gm, preferred_element_type=jnp.float32,
                    precision=jax.lax.Precision.HIGHEST)
    w = e / s
    vr = vr_ref[...]
    vt = jnp.concatenate([vr, vr, vr, vr], axis=1)
    ri = lax.broadcasted_iota(jnp.int32, (LV, L * D), 0) // V
    ci = lax.broadcasted_iota(jnp.int32, (LV, L * D), 1) // D
    bd = jnp.where(ri == ci, vt, 0.0)
    mm = lax.dot_general(bd, w_ref[...], (((1,), (1,)), ((), ())),
                         preferred_element_type=jnp.float32,
                         precision=jax.lax.Precision.HIGHEST)
    o_ref[...] = (
        jax.lax.dot(w, mm, preferred_element_type=jnp.float32,
                    precision=jax.lax.Precision.HIGHEST) + b_ref[...]
    )


def _tc_dense(g, vr, t_full, w, b2d):
    blk = 2048
    grid = (B // blk,)
    return pl.pallas_call(
        _dense_body,
        grid=grid,
        in_specs=[
            pl.BlockSpec((blk, LV), lambda i: (i, 0)),
            pl.BlockSpec((LV, D), lambda i: (0, 0)),
            pl.BlockSpec((1, LV), lambda i: (0, 0)),
            pl.BlockSpec((TD, L * D), lambda i: (0, 0)),
            pl.BlockSpec((1, TD), lambda i: (0, 0)),
        ],
        out_specs=pl.BlockSpec((blk, TD), lambda i: (i, 0)),
        out_shape=jax.ShapeDtypeStruct((B, TD), jnp.float32),
    )(g, vr, t_full, w, b2d)


def kernel(idx_batch, logits_table, vertices, logit_temps, W, b):
    info = plsc.get_sparse_core_info()
    nw = info.num_cores * info.num_subcores
    b_per_w = B // nw
    n_ch = b_per_w // CH

    tt3 = logits_table.transpose(1, 2, 0)
    t8 = _tc_transpose(tt3)
    table2d = t8.reshape(-1, LV)

    cq = TC_C // 8
    vo = idx_batch.astype(jnp.int32)
    row16 = ((vo // TC_C) * cq + vo % cq) * 8 + (vo % TC_C) // cq
    idx3d = row16.reshape(nw, n_ch, CH)
    g = _sc_gather(table2d, idx3d, n_ch, b_per_w, info.num_cores)

    vr = vertices.reshape(LV, D)
    t_full = jnp.repeat(logit_temps, V).reshape(1, LV)
    b2d = b.reshape(1, TD)
    return _tc_dense(g, vr, t_full, W, b2d)


# speedup vs baseline: 3.3637x; 1.4108x over previous
import functools

import jax
import jax.numpy as jnp
from jax import lax
from jax.experimental import pallas as pl
from jax.experimental.pallas import tpu as pltpu
from jax.experimental.pallas import tpu_sc as plsc

B = 16384
L = 4
V = 4
D = 8
TD = 64
LV = L * V
VOCAB = 1000000

CH = 128
TC_C = 8192


def _transpose_body(t_ref, o_ref):
    cq = TC_C // 8
    x = t_ref[...].reshape(LV, TC_C)
    lane = lax.broadcasted_iota(jnp.int32, (LV, TC_C), 1)
    gvo = pl.program_id(0) * TC_C + lane
    x = jnp.where(gvo < VOCAB, x, 0.0)
    jj = lax.broadcasted_iota(jnp.int32, (LV, 128), 0)
    mm = lax.broadcasted_iota(jnp.int32, (LV, 128), 1)
    hi = x.astype(jnp.bfloat16)
    lo = (x - hi.astype(jnp.float32)).astype(jnp.bfloat16)
    acc = None
    for u in range(8):
        eu = (mm == LV * u + jj).astype(jnp.bfloat16)
        d = None
        for part in (hi, lo):
            pu = part[:, u * cq:(u + 1) * cq]
            t = lax.dot_general(pu, eu, (((0,), (0,)), ((), ())),
                                preferred_element_type=jnp.float32)
            d = t if d is None else d + t
        acc = d if acc is None else acc + d
    o_ref[...] = acc


def _tc_transpose(tt3):
    n_steps = (VOCAB + TC_C - 1) // TC_C
    vpad = n_steps * TC_C
    return pl.pallas_call(
        _transpose_body,
        grid=(n_steps,),
        in_specs=[pl.BlockSpec((L, V, TC_C), lambda i: (0, 0, i))],
        out_specs=pl.BlockSpec((TC_C // 8, 128), lambda i: (i, 0)),
        out_shape=jax.ShapeDtypeStruct((vpad // 8, 128), jnp.float32),
    )(tt3)


def _sc_gather(table2d, idx3d, n_ch, b_per_w, nc):
    mesh = plsc.VectorSubcoreMesh(core_axis_name="c", subcore_axis_name="s")

    @functools.partial(
        pl.kernel,
        mesh=mesh,
        out_type=jax.ShapeDtypeStruct((B, LV), jnp.float32),
        scratch_types=[
            pltpu.VMEM((n_ch, CH), jnp.int32),
            pltpu.VMEM((b_per_w, LV), jnp.float32),
            pltpu.SemaphoreType.DMA,
        ],
        compiler_params=pltpu.CompilerParams(use_tc_tiling_on_sc=False),
    )
    def k(table_hbm, idx_hbm, out_hbm, idx_v, rows_v, sem):
        wid = lax.axis_index("s") * nc + lax.axis_index("c")
        base = wid * b_per_w
        pltpu.sync_copy(idx_hbm.at[wid], idx_v)
        copies = []
        for j in range(n_ch):
            copies.append(
                pltpu.async_copy(
                    table_hbm.at[idx_v.at[j]],
                    rows_v.at[pl.ds(j * CH, CH)],
                    sem,
                )
            )
        for c in copies:
            c.wait()
        pltpu.sync_copy(rows_v, out_hbm.at[pl.ds(base, b_per_w)])

    return k(table2d, idx3d)


def _dense_body(g_ref, vr_ref, t_ref, w_ref, b_ref, o_ref):
    x = g_ref[...] * t_ref[...]
    m = jnp.max(x, axis=1, keepdims=True)
    e = jnp.exp(x - m)
    ii = lax.broadcasted_iota(jnp.int32, (LV, LV), 0) // V
    jj = lax.broadcasted_iota(jnp.int32, (LV, LV), 1) // V
    gm = (ii == jj).astype(jnp.float32)
    s = jax.lax.dot(e, gm, preferred_element_type=jnp.float32,
                    precision=jax.lax.Precision.HIGHEST)
    w = e / s
    vr = vr_ref[...]
    vt = jnp.concatenate([vr, vr, vr, vr], axis=1)
    ri = lax.broadcasted_iota(jnp.int32, (LV, L * D), 0) // V
    ci = lax.broadcasted_iota(jnp.int32, (LV, L * D), 1) // D
    bd = jnp.where(ri == ci, vt, 0.0)
    mm = lax.dot_general(bd, w_ref[...], (((1,), (1,)), ((), ())),
                         preferred_element_type=jnp.float32,
                         precision=jax.lax.Precision.HIGHEST)
    o_ref[...] = (
        jax.lax.dot(w, mm, preferred_element_type=jnp.float32,
                    precision=jax.lax.Precision.HIGHEST) + b_ref[...]
    )


def _tc_dense(g, vr, t_full, w, b2d):
    blk = 2048
    grid = (B // blk,)
    return pl.pallas_call(
        _dense_body,
        grid=grid,
        in_specs=[
            pl.BlockSpec((blk, LV), lambda i: (i, 0)),
            pl.BlockSpec((LV, D), lambda i: (0, 0)),
            pl.BlockSpec((1, LV), lambda i: (0, 0)),
            pl.BlockSpec((TD, L * D), lambda i: (0, 0)),
            pl.BlockSpec((1, TD), lambda i: (0, 0)),
        ],
        out_specs=pl.BlockSpec((blk, TD), lambda i: (i, 0)),
        out_shape=jax.ShapeDtypeStruct((B, TD), jnp.float32),
    )(g, vr, t_full, w, b2d)


def kernel(idx_batch, logits_table, vertices, logit_temps, W, b):
    info = plsc.get_sparse_core_info()
    nw = info.num_cores * info.num_subcores
    b_per_w = B // nw
    n_ch = b_per_w // CH

    tt3 = logits_table.transpose(1, 2, 0)
    t8 = _tc_transpose(tt3)
    table2d = t8.reshape(-1, LV)

    cq = TC_C // 8
    vo = idx_batch.astype(jnp.int32)
    row16 = ((vo // TC_C) * cq + vo % cq) * 8 + (vo % TC_C) // cq
    idx3d = row16.reshape(nw, n_ch, CH)
    g = _sc_gather(table2d, idx3d, n_ch, b_per_w, info.num_cores)

    vr = vertices.reshape(LV, D)
    t_full = jnp.repeat(logit_temps, V).reshape(1, LV)
    b2d = b.reshape(1, TD)
    return _tc_dense(g, vr, t_full, W, b2d)


# speedup vs baseline: 3.6558x; 1.0868x over previous
import functools

import jax
import jax.numpy as jnp
from jax import lax
from jax.experimental import pallas as pl
from jax.experimental.pallas import tpu as pltpu
from jax.experimental.pallas import tpu_sc as plsc

B = 16384
L = 4
V = 4
D = 8
TD = 64
LV = L * V
VOCAB = 1000000

CH = 128
TC_C = 8192


def _transpose_body(t_ref, o_ref):
    cq = TC_C // 8
    x = t_ref[...].reshape(LV, TC_C)
    lane = lax.broadcasted_iota(jnp.int32, (LV, TC_C), 1)
    gvo = pl.program_id(0) * TC_C + lane
    x = jnp.where(gvo < VOCAB, x, 0.0)
    jj = lax.broadcasted_iota(jnp.int32, (LV, 128), 0)
    mm = lax.broadcasted_iota(jnp.int32, (LV, 128), 1)
    hi = x.astype(jnp.bfloat16)
    lo = (x - hi.astype(jnp.float32)).astype(jnp.bfloat16)
    acc = None
    for u in range(8):
        eu = (mm == LV * u + jj).astype(jnp.bfloat16)
        d = None
        for part in (hi, lo):
            pu = part[:, u * cq:(u + 1) * cq]
            t = lax.dot_general(pu, eu, (((0,), (0,)), ((), ())),
                                preferred_element_type=jnp.float32)
            d = t if d is None else d + t
        acc = d if acc is None else acc + d
    o_ref[...] = acc


def _tc_transpose(tt3):
    n_steps = (VOCAB + TC_C - 1) // TC_C
    vpad = n_steps * TC_C
    return pl.pallas_call(
        _transpose_body,
        grid=(n_steps,),
        in_specs=[pl.BlockSpec((L, V, TC_C), lambda i: (0, 0, i))],
        out_specs=pl.BlockSpec((TC_C // 8, 128), lambda i: (i, 0)),
        out_shape=jax.ShapeDtypeStruct((vpad // 8, 128), jnp.float32),
    )(tt3)


def _sc_gather(table2d, idx3d, n_ch, b_per_w, nc):
    mesh = plsc.VectorSubcoreMesh(core_axis_name="c", subcore_axis_name="s")

    @functools.partial(
        pl.kernel,
        mesh=mesh,
        out_type=jax.ShapeDtypeStruct((B, LV), jnp.float32),
        scratch_types=[
            pltpu.VMEM((n_ch, CH), jnp.int32),
            pltpu.VMEM((b_per_w, LV), jnp.float32),
            pltpu.SemaphoreType.DMA,
        ],
        compiler_params=pltpu.CompilerParams(use_tc_tiling_on_sc=False),
    )
    def k(table_hbm, idx_hbm, out_hbm, idx_v, rows_v, sem):
        wid = lax.axis_index("s") * nc + lax.axis_index("c")
        base = wid * b_per_w
        pltpu.sync_copy(idx_hbm.at[wid], idx_v)
        copies = []
        for j in range(n_ch):
            copies.append(
                pltpu.async_copy(
                    table_hbm.at[idx_v.at[j]],
                    rows_v.at[pl.ds(j * CH, CH)],
                    sem,
                )
            )
        for c in copies:
            c.wait()
        pltpu.sync_copy(rows_v, out_hbm.at[pl.ds(base, b_per_w)])

    return k(table2d, idx3d)


def _hilo_dot(a, bmat):
    hi = a.astype(jnp.bfloat16)
    lo = (a - hi.astype(jnp.float32)).astype(jnp.bfloat16)
    bb = bmat.astype(jnp.bfloat16)
    return (lax.dot_general(hi, bb, (((1,), (0,)), ((), ())),
                            preferred_element_type=jnp.float32)
            + lax.dot_general(lo, bb, (((1,), (0,)), ((), ())),
                              preferred_element_type=jnp.float32))


def _dense_body(g_ref, vr_ref, t_ref, w_ref, b_ref, o_ref):
    x = g_ref[...] * t_ref[...]
    m = jnp.max(x, axis=1, keepdims=True)
    e = jnp.exp(x - m)
    ii = lax.broadcasted_iota(jnp.int32, (LV, LV), 0) // V
    jj = lax.broadcasted_iota(jnp.int32, (LV, LV), 1) // V
    gm = (ii == jj).astype(jnp.float32)
    s = _hilo_dot(e, gm)
    w = e / s
    vr = vr_ref[...]
    vt = jnp.concatenate([vr, vr, vr, vr], axis=1)
    ri = lax.broadcasted_iota(jnp.int32, (LV, L * D), 0) // V
    ci = lax.broadcasted_iota(jnp.int32, (LV, L * D), 1) // D
    bd = jnp.where(ri == ci, vt, 0.0)
    mm = lax.dot_general(bd, w_ref[...], (((1,), (1,)), ((), ())),
                         preferred_element_type=jnp.float32,
                         precision=jax.lax.Precision.HIGHEST)
    o_ref[...] = _hilo_dot(w, mm) + b_ref[...]


def _tc_dense(g, vr, t_full, w, b2d):
    blk = 8192
    grid = (B // blk,)
    return pl.pallas_call(
        _dense_body,
        grid=grid,
        in_specs=[
            pl.BlockSpec((blk, LV), lambda i: (i, 0)),
            pl.BlockSpec((LV, D), lambda i: (0, 0)),
            pl.BlockSpec((1, LV), lambda i: (0, 0)),
            pl.BlockSpec((TD, L * D), lambda i: (0, 0)),
            pl.BlockSpec((1, TD), lambda i: (0, 0)),
        ],
        out_specs=pl.BlockSpec((blk, TD), lambda i: (i, 0)),
        out_shape=jax.ShapeDtypeStruct((B, TD), jnp.float32),
    )(g, vr, t_full, w, b2d)


def kernel(idx_batch, logits_table, vertices, logit_temps, W, b):
    info = plsc.get_sparse_core_info()
    nw = info.num_cores * info.num_subcores
    b_per_w = B // nw
    n_ch = b_per_w // CH

    tt3 = logits_table.transpose(1, 2, 0)
    t8 = _tc_transpose(tt3)
    table2d = t8.reshape(-1, LV)

    cq = TC_C // 8
    vo = idx_batch.astype(jnp.int32)
    row16 = ((vo // TC_C) * cq + vo % cq) * 8 + (vo % TC_C) // cq
    idx3d = row16.reshape(nw, n_ch, CH)
    g = _sc_gather(table2d, idx3d, n_ch, b_per_w, info.num_cores)

    vr = vertices.reshape(LV, D)
    t_full = jnp.repeat(logit_temps, V).reshape(1, LV)
    b2d = b.reshape(1, TD)
    return _tc_dense(g, vr, t_full, W, b2d)


# speedup vs baseline: 5.1090x; 1.3975x over previous
import functools

import jax
import jax.numpy as jnp
from jax import lax
from jax.experimental import pallas as pl
from jax.experimental.pallas import tpu as pltpu
from jax.experimental.pallas import tpu_sc as plsc

B = 16384
L = 4
V = 4
D = 8
TD = 64
LV = L * V
VOCAB = 1000000

CH = 128
TC_C = 8192


def _transpose_body(t_ref, o_ref):
    cq = TC_C // 8
    x = t_ref[...].reshape(LV, TC_C)
    lane = lax.broadcasted_iota(jnp.int32, (LV, TC_C), 1)
    gvo = pl.program_id(0) * TC_C + lane
    x = jnp.where(gvo < VOCAB, x, 0.0)
    hi = x.astype(jnp.bfloat16)
    lo = (x - hi.astype(jnp.float32)).astype(jnp.bfloat16)
    parts = [p[:, u * cq:(u + 1) * cq]
             for p in (hi, lo) for u in range(8)]
    xall = jnp.concatenate(parts, axis=0)
    kk = lax.broadcasted_iota(jnp.int32, (256, 128), 0)
    mm = lax.broadcasted_iota(jnp.int32, (256, 128), 1)
    ee = (kk % 128 == mm).astype(jnp.bfloat16)
    o_ref[...] = lax.dot_general(xall, ee, (((0,), (0,)), ((), ())),
                                 preferred_element_type=jnp.float32)


def _tc_transpose(tt3):
    n_steps = (VOCAB + TC_C - 1) // TC_C
    vpad = n_steps * TC_C
    return pl.pallas_call(
        _transpose_body,
        grid=(n_steps,),
        in_specs=[pl.BlockSpec((L, V, TC_C), lambda i: (0, 0, i))],
        out_specs=pl.BlockSpec((TC_C // 8, 128), lambda i: (i, 0)),
        out_shape=jax.ShapeDtypeStruct((vpad // 8, 128), jnp.float32),
    )(tt3)


def _sc_gather(table2d, idx3d, n_ch, b_per_w, nc):
    mesh = plsc.VectorSubcoreMesh(core_axis_name="c", subcore_axis_name="s")

    @functools.partial(
        pl.kernel,
        mesh=mesh,
        out_type=jax.ShapeDtypeStruct((B, LV), jnp.float32),
        scratch_types=[
            pltpu.VMEM((n_ch, CH), jnp.int32),
            pltpu.VMEM((b_per_w, LV), jnp.float32),
            pltpu.SemaphoreType.DMA,
        ],
        compiler_params=pltpu.CompilerParams(use_tc_tiling_on_sc=False),
    )
    def k(table_hbm, idx_hbm, out_hbm, idx_v, rows_v, sem):
        wid = lax.axis_index("s") * nc + lax.axis_index("c")
        base = wid * b_per_w
        pltpu.sync_copy(idx_hbm.at[wid], idx_v)
        copies = []
        for j in range(n_ch):
            copies.append(
                pltpu.async_copy(
                    table_hbm.at[idx_v.at[j]],
                    rows_v.at[pl.ds(j * CH, CH)],
                    sem,
                )
            )
        for c in copies:
            c.wait()
        pltpu.sync_copy(rows_v, out_hbm.at[pl.ds(base, b_per_w)])

    return k(table2d, idx3d)


def _hilo_dot(a, bmat):
    hi = a.astype(jnp.bfloat16)
    lo = (a - hi.astype(jnp.float32)).astype(jnp.bfloat16)
    bb = bmat.astype(jnp.bfloat16)
    return (lax.dot_general(hi, bb, (((1,), (0,)), ((), ())),
                            preferred_element_type=jnp.float32)
            + lax.dot_general(lo, bb, (((1,), (0,)), ((), ())),
                              preferred_element_type=jnp.float32))


def _dense_body(g_ref, vr_ref, t_ref, w_ref, b_ref, o_ref):
    x = g_ref[...] * t_ref[...]
    m = jnp.max(x, axis=1, keepdims=True)
    e = jnp.exp(x - m)
    ii = lax.broadcasted_iota(jnp.int32, (LV, LV), 0) // V
    jj = lax.broadcasted_iota(jnp.int32, (LV, LV), 1) // V
    gm = (ii == jj).astype(jnp.float32)
    s = _hilo_dot(e, gm)
    w = e / s
    vr = vr_ref[...]
    vt = jnp.concatenate([vr, vr, vr, vr], axis=1)
    ri = lax.broadcasted_iota(jnp.int32, (LV, L * D), 0) // V
    ci = lax.broadcasted_iota(jnp.int32, (LV, L * D), 1) // D
    bd = jnp.where(ri == ci, vt, 0.0)
    mm = lax.dot_general(bd, w_ref[...], (((1,), (1,)), ((), ())),
                         preferred_element_type=jnp.float32,
                         precision=jax.lax.Precision.HIGHEST)
    o_ref[...] = _hilo_dot(w, mm) + b_ref[...]


def _tc_dense(g, vr, t_full, w, b2d):
    blk = 8192
    grid = (B // blk,)
    return pl.pallas_call(
        _dense_body,
        grid=grid,
        in_specs=[
            pl.BlockSpec((blk, LV), lambda i: (i, 0)),
            pl.BlockSpec((LV, D), lambda i: (0, 0)),
            pl.BlockSpec((1, LV), lambda i: (0, 0)),
            pl.BlockSpec((TD, L * D), lambda i: (0, 0)),
            pl.BlockSpec((1, TD), lambda i: (0, 0)),
        ],
        out_specs=pl.BlockSpec((blk, TD), lambda i: (i, 0)),
        out_shape=jax.ShapeDtypeStruct((B, TD), jnp.float32),
    )(g, vr, t_full, w, b2d)


def kernel(idx_batch, logits_table, vertices, logit_temps, W, b):
    info = plsc.get_sparse_core_info()
    nw = info.num_cores * info.num_subcores
    b_per_w = B // nw
    n_ch = b_per_w // CH

    tt3 = logits_table.transpose(1, 2, 0)
    t8 = _tc_transpose(tt3)
    table2d = t8.reshape(-1, LV)

    cq = TC_C // 8
    vo = idx_batch.astype(jnp.int32)
    row16 = ((vo // TC_C) * cq + vo % cq) * 8 + (vo % TC_C) // cq
    idx3d = row16.reshape(nw, n_ch, CH)
    g = _sc_gather(table2d, idx3d, n_ch, b_per_w, info.num_cores)

    vr = vertices.reshape(LV, D)
    t_full = jnp.repeat(logit_temps, V).reshape(1, LV)
    b2d = b.reshape(1, TD)
    return _tc_dense(g, vr, t_full, W, b2d)


# speedup vs baseline: 6.5266x; 1.2775x over previous
import functools

import jax
import jax.numpy as jnp
from jax import lax
from jax.experimental import pallas as pl
from jax.experimental.pallas import tpu as pltpu
from jax.experimental.pallas import tpu_sc as plsc

B = 16384
L = 4
V = 4
D = 8
TD = 64
LV = L * V
VOCAB = 1000000

CH = 128
TC_C = 16384


def _transpose_body(t_ref, o_ref):
    cq = TC_C // 8
    x = t_ref[...].reshape(LV, TC_C)
    lane = lax.broadcasted_iota(jnp.int32, (LV, TC_C), 1)
    gvo = pl.program_id(0) * TC_C + lane
    x = jnp.where(gvo < VOCAB, x, 0.0)
    hi = x.astype(jnp.bfloat16)
    lo = (x - hi.astype(jnp.float32)).astype(jnp.bfloat16)
    parts = [p[:, u * cq:(u + 1) * cq]
             for p in (hi, lo) for u in range(8)]
    xall = jnp.concatenate(parts, axis=0)
    kk = lax.broadcasted_iota(jnp.int32, (256, 128), 0)
    mm = lax.broadcasted_iota(jnp.int32, (256, 128), 1)
    ee = (kk % 128 == mm).astype(jnp.bfloat16)
    o_ref[...] = lax.dot_general(xall, ee, (((0,), (0,)), ((), ())),
                                 preferred_element_type=jnp.float32)


def _tc_transpose(tt3):
    n_steps = (VOCAB + TC_C - 1) // TC_C
    vpad = n_steps * TC_C
    return pl.pallas_call(
        _transpose_body,
        grid=(n_steps,),
        in_specs=[pl.BlockSpec((L, V, TC_C), lambda i: (0, 0, i))],
        out_specs=pl.BlockSpec((TC_C // 8, 128), lambda i: (i, 0)),
        out_shape=jax.ShapeDtypeStruct((vpad // 8, 128), jnp.float32),
    )(tt3)


def _sc_gather(table2d, idx3d, n_ch, b_per_w, nc):
    mesh = plsc.VectorSubcoreMesh(core_axis_name="c", subcore_axis_name="s")

    @functools.partial(
        pl.kernel,
        mesh=mesh,
        out_type=jax.ShapeDtypeStruct((B, LV), jnp.float32),
        scratch_types=[
            pltpu.VMEM((n_ch, CH), jnp.int32),
            pltpu.VMEM((b_per_w, LV), jnp.float32),
            pltpu.SemaphoreType.DMA,
        ],
        compiler_params=pltpu.CompilerParams(use_tc_tiling_on_sc=False),
    )
    def k(table_hbm, idx_hbm, out_hbm, idx_v, rows_v, sem):
        wid = lax.axis_index("s") * nc + lax.axis_index("c")
        base = wid * b_per_w
        pltpu.sync_copy(idx_hbm.at[wid], idx_v)
        copies = []
        for j in range(n_ch):
            copies.append(
                pltpu.async_copy(
                    table_hbm.at[idx_v.at[j]],
                    rows_v.at[pl.ds(j * CH, CH)],
                    sem,
                )
            )
        for c in copies:
            c.wait()
        pltpu.sync_copy(rows_v, out_hbm.at[pl.ds(base, b_per_w)])

    return k(table2d, idx3d)


def _hilo_dot(a, bmat):
    hi = a.astype(jnp.bfloat16)
    lo = (a - hi.astype(jnp.float32)).astype(jnp.bfloat16)
    bb = bmat.astype(jnp.bfloat16)
    return (lax.dot_general(hi, bb, (((1,), (0,)), ((), ())),
                            preferred_element_type=jnp.float32)
            + lax.dot_general(lo, bb, (((1,), (0,)), ((), ())),
                              preferred_element_type=jnp.float32))


def _dense_body(g_ref, vr_ref, t_ref, w_ref, b_ref, o_ref):
    x = g_ref[...] * t_ref[...]
    m = jnp.max(x, axis=1, keepdims=True)
    e = jnp.exp(x - m)
    ii = lax.broadcasted_iota(jnp.int32, (LV, LV), 0) // V
    jj = lax.broadcasted_iota(jnp.int32, (LV, LV), 1) // V
    gm = (ii == jj).astype(jnp.float32)
    s = _hilo_dot(e, gm)
    w = e / s
    vr = vr_ref[...]
    vt = jnp.concatenate([vr, vr, vr, vr], axis=1)
    ri = lax.broadcasted_iota(jnp.int32, (LV, L * D), 0) // V
    ci = lax.broadcasted_iota(jnp.int32, (LV, L * D), 1) // D
    bd = jnp.where(ri == ci, vt, 0.0)
    mm = lax.dot_general(bd, w_ref[...], (((1,), (1,)), ((), ())),
                         preferred_element_type=jnp.float32,
                         precision=jax.lax.Precision.HIGHEST)
    o_ref[...] = _hilo_dot(w, mm) + b_ref[...]


def _tc_dense(g, vr, t_full, w, b2d):
    blk = 8192
    grid = (B // blk,)
    return pl.pallas_call(
        _dense_body,
        grid=grid,
        in_specs=[
            pl.BlockSpec((blk, LV), lambda i: (i, 0)),
            pl.BlockSpec((LV, D), lambda i: (0, 0)),
            pl.BlockSpec((1, LV), lambda i: (0, 0)),
            pl.BlockSpec((TD, L * D), lambda i: (0, 0)),
            pl.BlockSpec((1, TD), lambda i: (0, 0)),
        ],
        out_specs=pl.BlockSpec((blk, TD), lambda i: (i, 0)),
        out_shape=jax.ShapeDtypeStruct((B, TD), jnp.float32),
    )(g, vr, t_full, w, b2d)


def kernel(idx_batch, logits_table, vertices, logit_temps, W, b):
    info = plsc.get_sparse_core_info()
    nw = info.num_cores * info.num_subcores
    b_per_w = B // nw
    n_ch = b_per_w // CH

    tt3 = logits_table.transpose(1, 2, 0)
    t8 = _tc_transpose(tt3)
    table2d = t8.reshape(-1, LV)

    cq = TC_C // 8
    vo = idx_batch.astype(jnp.int32)
    row16 = ((vo // TC_C) * cq + vo % cq) * 8 + (vo % TC_C) // cq
    idx3d = row16.reshape(nw, n_ch, CH)
    g = _sc_gather(table2d, idx3d, n_ch, b_per_w, info.num_cores)

    vr = vertices.reshape(LV, D)
    t_full = jnp.repeat(logit_temps, V).reshape(1, LV)
    b2d = b.reshape(1, TD)
    return _tc_dense(g, vr, t_full, W, b2d)


# speedup vs baseline: 7.8417x; 1.2015x over previous
import functools

import jax
import jax.numpy as jnp
from jax import lax
from jax.experimental import pallas as pl
from jax.experimental.pallas import tpu as pltpu
from jax.experimental.pallas import tpu_sc as plsc

B = 16384
L = 4
V = 4
D = 8
TD = 64
LV = L * V
VOCAB = 1000000

CH = 128
TC_C = 32768


def _transpose_body(t_ref, o_ref):
    cq = TC_C // 8
    x = t_ref[...].reshape(LV, TC_C)
    lane = lax.broadcasted_iota(jnp.int32, (LV, TC_C), 1)
    gvo = pl.program_id(0) * TC_C + lane
    x = jnp.where(gvo < VOCAB, x, 0.0)
    hi = x.astype(jnp.bfloat16)
    lo = (x - hi.astype(jnp.float32)).astype(jnp.bfloat16)
    parts = [p[:, u * cq:(u + 1) * cq]
             for p in (hi, lo) for u in range(8)]
    xall = jnp.concatenate(parts, axis=0)
    kk = lax.broadcasted_iota(jnp.int32, (256, 128), 0)
    mm = lax.broadcasted_iota(jnp.int32, (256, 128), 1)
    ee = (kk % 128 == mm).astype(jnp.bfloat16)
    o_ref[...] = lax.dot_general(xall, ee, (((0,), (0,)), ((), ())),
                                 preferred_element_type=jnp.float32)


def _tc_transpose(tt3):
    n_steps = (VOCAB + TC_C - 1) // TC_C
    vpad = n_steps * TC_C
    return pl.pallas_call(
        _transpose_body,
        grid=(n_steps,),
        in_specs=[pl.BlockSpec((L, V, TC_C), lambda i: (0, 0, i))],
        out_specs=pl.BlockSpec((TC_C // 8, 128), lambda i: (i, 0)),
        out_shape=jax.ShapeDtypeStruct((vpad // 8, 128), jnp.float32),
    )(tt3)


def _sc_gather(table2d, idx3d, n_ch, b_per_w, nc):
    mesh = plsc.VectorSubcoreMesh(core_axis_name="c", subcore_axis_name="s")

    @functools.partial(
        pl.kernel,
        mesh=mesh,
        out_type=jax.ShapeDtypeStruct((B, LV), jnp.float32),
        scratch_types=[
            pltpu.VMEM((n_ch, CH), jnp.int32),
            pltpu.VMEM((b_per_w, LV), jnp.float32),
            pltpu.SemaphoreType.DMA,
        ],
        compiler_params=pltpu.CompilerParams(use_tc_tiling_on_sc=False),
    )
    def k(table_hbm, idx_hbm, out_hbm, idx_v, rows_v, sem):
        wid = lax.axis_index("s") * nc + lax.axis_index("c")
        base = wid * b_per_w
        pltpu.sync_copy(idx_hbm.at[wid], idx_v)
        copies = []
        for j in range(n_ch):
            copies.append(
                pltpu.async_copy(
                    table_hbm.at[idx_v.at[j]],
                    rows_v.at[pl.ds(j * CH, CH)],
                    sem,
                )
            )
        for c in copies:
            c.wait()
        pltpu.sync_copy(rows_v, out_hbm.at[pl.ds(base, b_per_w)])

    return k(table2d, idx3d)


def _hilo_dot(a, bmat):
    hi = a.astype(jnp.bfloat16)
    lo = (a - hi.astype(jnp.float32)).astype(jnp.bfloat16)
    bb = bmat.astype(jnp.bfloat16)
    return (lax.dot_general(hi, bb, (((1,), (0,)), ((), ())),
                            preferred_element_type=jnp.float32)
            + lax.dot_general(lo, bb, (((1,), (0,)), ((), ())),
                              preferred_element_type=jnp.float32))


def _dense_body(g_ref, vr_ref, t_ref, w_ref, b_ref, o_ref):
    x = g_ref[...] * t_ref[...]
    m = jnp.max(x, axis=1, keepdims=True)
    e = jnp.exp(x - m)
    ii = lax.broadcasted_iota(jnp.int32, (LV, LV), 0) // V
    jj = lax.broadcasted_iota(jnp.int32, (LV, LV), 1) // V
    gm = (ii == jj).astype(jnp.float32)
    s = _hilo_dot(e, gm)
    w = e / s
    vr = vr_ref[...]
    vt = jnp.concatenate([vr, vr, vr, vr], axis=1)
    ri = lax.broadcasted_iota(jnp.int32, (LV, L * D), 0) // V
    ci = lax.broadcasted_iota(jnp.int32, (LV, L * D), 1) // D
    bd = jnp.where(ri == ci, vt, 0.0)
    mm = lax.dot_general(bd, w_ref[...], (((1,), (1,)), ((), ())),
                         preferred_element_type=jnp.float32,
                         precision=jax.lax.Precision.HIGHEST)
    o_ref[...] = _hilo_dot(w, mm) + b_ref[...]


def _tc_dense(g, vr, t_full, w, b2d):
    blk = 8192
    grid = (B // blk,)
    return pl.pallas_call(
        _dense_body,
        grid=grid,
        in_specs=[
            pl.BlockSpec((blk, LV), lambda i: (i, 0)),
            pl.BlockSpec((LV, D), lambda i: (0, 0)),
            pl.BlockSpec((1, LV), lambda i: (0, 0)),
            pl.BlockSpec((TD, L * D), lambda i: (0, 0)),
            pl.BlockSpec((1, TD), lambda i: (0, 0)),
        ],
        out_specs=pl.BlockSpec((blk, TD), lambda i: (i, 0)),
        out_shape=jax.ShapeDtypeStruct((B, TD), jnp.float32),
    )(g, vr, t_full, w, b2d)


def kernel(idx_batch, logits_table, vertices, logit_temps, W, b):
    info = plsc.get_sparse_core_info()
    nw = info.num_cores * info.num_subcores
    b_per_w = B // nw
    n_ch = b_per_w // CH

    tt3 = logits_table.transpose(1, 2, 0)
    t8 = _tc_transpose(tt3)
    table2d = t8.reshape(-1, LV)

    cq = TC_C // 8
    vo = idx_batch.astype(jnp.int32)
    row16 = ((vo // TC_C) * cq + vo % cq) * 8 + (vo % TC_C) // cq
    idx3d = row16.reshape(nw, n_ch, CH)
    g = _sc_gather(table2d, idx3d, n_ch, b_per_w, info.num_cores)

    vr = vertices.reshape(LV, D)
    t_full = jnp.repeat(logit_temps, V).reshape(1, LV)
    b2d = b.reshape(1, TD)
    return _tc_dense(g, vr, t_full, W, b2d)


# speedup vs baseline: 8.6018x; 1.0969x over previous
import functools

import jax
import jax.numpy as jnp
from jax import lax
from jax.experimental import pallas as pl
from jax.experimental.pallas import tpu as pltpu
from jax.experimental.pallas import tpu_sc as plsc

B = 16384
L = 4
V = 4
D = 8
TD = 64
LV = L * V
VOCAB = 1000000

CH = 128
TC_C = 65536


def _transpose_body(t_ref, o_ref):
    cq = TC_C // 8
    x = t_ref[...].reshape(LV, TC_C)
    lane = lax.broadcasted_iota(jnp.int32, (LV, TC_C), 1)
    gvo = pl.program_id(0) * TC_C + lane
    x = jnp.where(gvo < VOCAB, x, 0.0)
    hi = x.astype(jnp.bfloat16)
    lo = (x - hi.astype(jnp.float32)).astype(jnp.bfloat16)
    parts = [p[:, u * cq:(u + 1) * cq]
             for p in (hi, lo) for u in range(8)]
    xall = jnp.concatenate(parts, axis=0)
    kk = lax.broadcasted_iota(jnp.int32, (256, 128), 0)
    mm = lax.broadcasted_iota(jnp.int32, (256, 128), 1)
    ee = (kk % 128 == mm).astype(jnp.bfloat16)
    o_ref[...] = lax.dot_general(xall, ee, (((0,), (0,)), ((), ())),
                                 preferred_element_type=jnp.float32)


def _tc_transpose(tt3):
    n_steps = (VOCAB + TC_C - 1) // TC_C
    vpad = n_steps * TC_C
    return pl.pallas_call(
        _transpose_body,
        grid=(n_steps,),
        in_specs=[pl.BlockSpec((L, V, TC_C), lambda i: (0, 0, i))],
        out_specs=pl.BlockSpec((TC_C // 8, 128), lambda i: (i, 0)),
        out_shape=jax.ShapeDtypeStruct((vpad // 8, 128), jnp.float32),
    )(tt3)


def _sc_gather(table2d, idx3d, n_ch, b_per_w, nc):
    mesh = plsc.VectorSubcoreMesh(core_axis_name="c", subcore_axis_name="s")

    @functools.partial(
        pl.kernel,
        mesh=mesh,
        out_type=jax.ShapeDtypeStruct((B, LV), jnp.float32),
        scratch_types=[
            pltpu.VMEM((n_ch, CH), jnp.int32),
            pltpu.VMEM((b_per_w, LV), jnp.float32),
            pltpu.SemaphoreType.DMA,
        ],
        compiler_params=pltpu.CompilerParams(use_tc_tiling_on_sc=False),
    )
    def k(table_hbm, idx_hbm, out_hbm, idx_v, rows_v, sem):
        wid = lax.axis_index("s") * nc + lax.axis_index("c")
        base = wid * b_per_w
        pltpu.sync_copy(idx_hbm.at[wid], idx_v)
        copies = []
        for j in range(n_ch):
            copies.append(
                pltpu.async_copy(
                    table_hbm.at[idx_v.at[j]],
                    rows_v.at[pl.ds(j * CH, CH)],
                    sem,
                )
            )
        for c in copies:
            c.wait()
        pltpu.sync_copy(rows_v, out_hbm.at[pl.ds(base, b_per_w)])

    return k(table2d, idx3d)


def _hilo_dot(a, bmat):
    hi = a.astype(jnp.bfloat16)
    lo = (a - hi.astype(jnp.float32)).astype(jnp.bfloat16)
    bb = bmat.astype(jnp.bfloat16)
    return (lax.dot_general(hi, bb, (((1,), (0,)), ((), ())),
                            preferred_element_type=jnp.float32)
            + lax.dot_general(lo, bb, (((1,), (0,)), ((), ())),
                              preferred_element_type=jnp.float32))


def _dense_body(g_ref, vr_ref, t_ref, w_ref, b_ref, o_ref):
    x = g_ref[...] * t_ref[...]
    m = jnp.max(x, axis=1, keepdims=True)
    e = jnp.exp(x - m)
    ii = lax.broadcasted_iota(jnp.int32, (LV, LV), 0) // V
    jj = lax.broadcasted_iota(jnp.int32, (LV, LV), 1) // V
    gm = (ii == jj).astype(jnp.float32)
    s = _hilo_dot(e, gm)
    w = e / s
    vr = vr_ref[...]
    vt = jnp.concatenate([vr, vr, vr, vr], axis=1)
    ri = lax.broadcasted_iota(jnp.int32, (LV, L * D), 0) // V
    ci = lax.broadcasted_iota(jnp.int32, (LV, L * D), 1) // D
    bd = jnp.where(ri == ci, vt, 0.0)
    mm = lax.dot_general(bd, w_ref[...], (((1,), (1,)), ((), ())),
                         preferred_element_type=jnp.float32,
                         precision=jax.lax.Precision.HIGHEST)
    o_ref[...] = _hilo_dot(w, mm) + b_ref[...]


def _tc_dense(g, vr, t_full, w, b2d):
    blk = 8192
    grid = (B // blk,)
    return pl.pallas_call(
        _dense_body,
        grid=grid,
        in_specs=[
            pl.BlockSpec((blk, LV), lambda i: (i, 0)),
            pl.BlockSpec((LV, D), lambda i: (0, 0)),
            pl.BlockSpec((1, LV), lambda i: (0, 0)),
            pl.BlockSpec((TD, L * D), lambda i: (0, 0)),
            pl.BlockSpec((1, TD), lambda i: (0, 0)),
        ],
        out_specs=pl.BlockSpec((blk, TD), lambda i: (i, 0)),
        out_shape=jax.ShapeDtypeStruct((B, TD), jnp.float32),
    )(g, vr, t_full, w, b2d)


def kernel(idx_batch, logits_table, vertices, logit_temps, W, b):
    info = plsc.get_sparse_core_info()
    nw = info.num_cores * info.num_subcores
    b_per_w = B // nw
    n_ch = b_per_w // CH

    tt3 = logits_table.transpose(1, 2, 0)
    t8 = _tc_transpose(tt3)
    table2d = t8.reshape(-1, LV)

    cq = TC_C // 8
    vo = idx_batch.astype(jnp.int32)
    row16 = ((vo // TC_C) * cq + vo % cq) * 8 + (vo % TC_C) // cq
    idx3d = row16.reshape(nw, n_ch, CH)
    g = _sc_gather(table2d, idx3d, n_ch, b_per_w, info.num_cores)

    vr = vertices.reshape(LV, D)
    t_full = jnp.repeat(logit_temps, V).reshape(1, LV)
    b2d = b.reshape(1, TD)
    return _tc_dense(g, vr, t_full, W, b2d)


# speedup vs baseline: 8.7809x; 1.0208x over previous
import functools

import jax
import jax.numpy as jnp
from jax import lax
from jax.experimental import pallas as pl
from jax.experimental.pallas import tpu as pltpu
from jax.experimental.pallas import tpu_sc as plsc

B = 16384
L = 4
V = 4
D = 8
TD = 64
LV = L * V
VOCAB = 1000000

CH = 128
TC_C = 131072


def _transpose_body(t_ref, o_ref):
    cq = TC_C // 8
    x = t_ref[...].reshape(LV, TC_C)
    lane = lax.broadcasted_iota(jnp.int32, (LV, TC_C), 1)
    gvo = pl.program_id(0) * TC_C + lane
    x = jnp.where(gvo < VOCAB, x, 0.0)
    hi = x.astype(jnp.bfloat16)
    lo = (x - hi.astype(jnp.float32)).astype(jnp.bfloat16)
    parts = [p[:, u * cq:(u + 1) * cq]
             for p in (hi, lo) for u in range(8)]
    xall = jnp.concatenate(parts, axis=0)
    kk = lax.broadcasted_iota(jnp.int32, (256, 128), 0)
    mm = lax.broadcasted_iota(jnp.int32, (256, 128), 1)
    ee = (kk % 128 == mm).astype(jnp.bfloat16)
    o_ref[...] = lax.dot_general(xall, ee, (((0,), (0,)), ((), ())),
                                 preferred_element_type=jnp.float32)


def _tc_transpose(tt3):
    n_steps = (VOCAB + TC_C - 1) // TC_C
    vpad = n_steps * TC_C
    return pl.pallas_call(
        _transpose_body,
        grid=(n_steps,),
        in_specs=[pl.BlockSpec((L, V, TC_C), lambda i: (0, 0, i))],
        out_specs=pl.BlockSpec((TC_C // 8, 128), lambda i: (i, 0)),
        out_shape=jax.ShapeDtypeStruct((vpad // 8, 128), jnp.float32),
    )(tt3)


def _sc_gather(table2d, idx3d, n_ch, b_per_w, nc):
    mesh = plsc.VectorSubcoreMesh(core_axis_name="c", subcore_axis_name="s")

    @functools.partial(
        pl.kernel,
        mesh=mesh,
        out_type=jax.ShapeDtypeStruct((B, LV), jnp.float32),
        scratch_types=[
            pltpu.VMEM((n_ch, CH), jnp.int32),
            pltpu.VMEM((b_per_w, LV), jnp.float32),
            pltpu.SemaphoreType.DMA,
        ],
        compiler_params=pltpu.CompilerParams(use_tc_tiling_on_sc=False),
    )
    def k(table_hbm, idx_hbm, out_hbm, idx_v, rows_v, sem):
        wid = lax.axis_index("s") * nc + lax.axis_index("c")
        base = wid * b_per_w
        pltpu.sync_copy(idx_hbm.at[wid], idx_v)
        copies = []
        for j in range(n_ch):
            copies.append(
                pltpu.async_copy(
                    table_hbm.at[idx_v.at[j]],
                    rows_v.at[pl.ds(j * CH, CH)],
                    sem,
                )
            )
        for c in copies:
            c.wait()
        pltpu.sync_copy(rows_v, out_hbm.at[pl.ds(base, b_per_w)])

    return k(table2d, idx3d)


def _hilo_dot(a, bmat):
    hi = a.astype(jnp.bfloat16)
    lo = (a - hi.astype(jnp.float32)).astype(jnp.bfloat16)
    bb = bmat.astype(jnp.bfloat16)
    return (lax.dot_general(hi, bb, (((1,), (0,)), ((), ())),
                            preferred_element_type=jnp.float32)
            + lax.dot_general(lo, bb, (((1,), (0,)), ((), ())),
                              preferred_element_type=jnp.float32))


def _dense_body(g_ref, vr_ref, t_ref, w_ref, b_ref, o_ref):
    x = g_ref[...] * t_ref[...]
    m = jnp.max(x, axis=1, keepdims=True)
    e = jnp.exp(x - m)
    ii = lax.broadcasted_iota(jnp.int32, (LV, LV), 0) // V
    jj = lax.broadcasted_iota(jnp.int32, (LV, LV), 1) // V
    gm = (ii == jj).astype(jnp.float32)
    s = _hilo_dot(e, gm)
    w = e / s
    vr = vr_ref[...]
    vt = jnp.concatenate([vr, vr, vr, vr], axis=1)
    ri = lax.broadcasted_iota(jnp.int32, (LV, L * D), 0) // V
    ci = lax.broadcasted_iota(jnp.int32, (LV, L * D), 1) // D
    bd = jnp.where(ri == ci, vt, 0.0)
    mm = lax.dot_general(bd, w_ref[...], (((1,), (1,)), ((), ())),
                         preferred_element_type=jnp.float32,
                         precision=jax.lax.Precision.HIGHEST)
    o_ref[...] = _hilo_dot(w, mm) + b_ref[...]


def _tc_dense(g, vr, t_full, w, b2d):
    blk = 8192
    grid = (B // blk,)
    return pl.pallas_call(
        _dense_body,
        grid=grid,
        in_specs=[
            pl.BlockSpec((blk, LV), lambda i: (i, 0)),
            pl.BlockSpec((LV, D), lambda i: (0, 0)),
            pl.BlockSpec((1, LV), lambda i: (0, 0)),
            pl.BlockSpec((TD, L * D), lambda i: (0, 0)),
            pl.BlockSpec((1, TD), lambda i: (0, 0)),
        ],
        out_specs=pl.BlockSpec((blk, TD), lambda i: (i, 0)),
        out_shape=jax.ShapeDtypeStruct((B, TD), jnp.float32),
    )(g, vr, t_full, w, b2d)


def kernel(idx_batch, logits_table, vertices, logit_temps, W, b):
    info = plsc.get_sparse_core_info()
    nw = info.num_cores * info.num_subcores
    b_per_w = B // nw
    n_ch = b_per_w // CH

    tt3 = logits_table.transpose(1, 2, 0)
    t8 = _tc_transpose(tt3)
    table2d = t8.reshape(-1, LV)

    cq = TC_C // 8
    vo = idx_batch.astype(jnp.int32)
    row16 = ((vo // TC_C) * cq + vo % cq) * 8 + (vo % TC_C) // cq
    idx3d = row16.reshape(nw, n_ch, CH)
    g = _sc_gather(table2d, idx3d, n_ch, b_per_w, info.num_cores)

    vr = vertices.reshape(LV, D)
    t_full = jnp.repeat(logit_temps, V).reshape(1, LV)
    b2d = b.reshape(1, TD)
    return _tc_dense(g, vr, t_full, W, b2d)
